# baseline (speedup 1.0000x reference)
import functools

import jax
import jax.numpy as jnp
from jax import lax
from jax.experimental import pallas as pl
from jax.experimental.pallas import tpu as pltpu
from jax.experimental.pallas import tpu_sc as plsc

N_VOX = 100000
M = 20000
P = 16
TW = 128
NWORK = 32
SLAB = 8192
DENSE_ROWS = NWORK * SLAB
NPIX = 2 * 360 * 360
PAD_PER_W = 64
POOLED_ROWS = M + NWORK * PAD_PER_W
PFN_GRID = 23

VSX = 150.4 / 1440.0
VSZ = 6.0 / 40.0
VNX = 150.4 / 360.0
PMIN = -75.2
PMINZ = -2.0


def _table_kernel(vf_ref, zyx_ref, t_ref):
    vf = vf_ref[...]
    z = (zyx_ref[:, 0:1] % 40).astype(jnp.float32)
    y = zyx_ref[:, 1:2].astype(jnp.float32)
    x = zyx_ref[:, 2:3].astype(jnp.float32)
    cx = (x + 0.5) * VSX + PMIN
    cy = (y + 0.5) * VSX + PMIN
    cz = (z + 0.5) * VSZ + PMINZ
    pad = jnp.zeros((vf.shape[0], TW - 67), jnp.float32)
    t_ref[...] = jnp.concatenate([vf, cx, cy, cz, pad], axis=1)


def _build_table(voxel_features, sp_zyx):
    blk = 4000
    grid = N_VOX // blk
    return pl.pallas_call(
        _table_kernel,
        grid=(grid,),
        in_specs=[
            pl.BlockSpec((blk, 64), lambda i: (i, 0)),
            pl.BlockSpec((blk, 3), lambda i: (i, 0)),
        ],
        out_specs=pl.BlockSpec((blk, TW), lambda i: (i, 0)),
        out_shape=jax.ShapeDtypeStruct((N_VOX, TW), jnp.float32),
    )(voxel_features, sp_zyx)


def _make_gather():
    rows_per_w = (M * P) // NWORK
    chunk = 200
    npair = rows_per_w // (2 * chunk)
    mesh = plsc.VectorSubcoreMesh(core_axis_name="c", subcore_axis_name="s")

    @functools.partial(
        pl.kernel, mesh=mesh,
        out_type=jax.ShapeDtypeStruct((M * P, TW), jnp.float32),
        scratch_types=[
            pltpu.VMEM((rows_per_w,), jnp.int32),
            pltpu.VMEM((chunk, TW), jnp.float32),
            pltpu.VMEM((chunk, TW), jnp.float32),
            pltpu.SemaphoreType.DMA,
            pltpu.SemaphoreType.DMA,
            pltpu.SemaphoreType.DMA,
            pltpu.SemaphoreType.DMA,
        ],
    )
    def k(table_hbm, idx_hbm, out_hbm, idx_v, buf0, buf1, g0, g1, w0, w1):
        wid = lax.axis_index("s") * 2 + lax.axis_index("c")
        base = wid * rows_per_w
        pltpu.sync_copy(idx_hbm.at[pl.ds(base, rows_per_w)], idx_v)

        def body(i, _):
            c0 = base + 2 * i * chunk
            c1 = c0 + chunk
            h0 = pltpu.async_copy(
                table_hbm.at[idx_v.at[pl.ds(2 * i * chunk, chunk)]], buf0, g0)
            h1 = pltpu.async_copy(
                table_hbm.at[idx_v.at[pl.ds((2 * i + 1) * chunk, chunk)]],
                buf1, g1)
            h0.wait()
            o0 = pltpu.async_copy(buf0, out_hbm.at[pl.ds(c0, chunk)], w0)
            h1.wait()
            o1 = pltpu.async_copy(buf1, out_hbm.at[pl.ds(c1, chunk)], w1)
            o0.wait()
            o1.wait()
            return _

        lax.fori_loop(0, npair, body, 0)

    return k


def _pfn_kernel(g_ref, yx_ref, nb_ref, ny_ref, nx_ref,
                w1_ref, w1b_ref, b1_ref, w2a_ref, w2b_ref, b2_ref,
                pooled_ref, keys_ref):
    pid = pl.program_id(0)
    q = g_ref.shape[1]

    b_row = nb_ref[0]
    y_row = ny_ref[0]
    x_row = nx_ref[0]
    keys_ref[0] = (b_row * 360 + y_row) * 360 + x_row

    @pl.when(pid < 20)
    def _compute():
        qx = (yx_ref[:, 1:2].astype(jnp.float32) + 0.5) * VNX + PMIN
        qy = (yx_ref[:, 0:1].astype(jnp.float32) + 0.5) * VNX + PMIN
        ccomb = (qx * w1b_ref[0:1, :] + qy * w1b_ref[1:2, :]
                 + ((0.5 * 6.0 + PMINZ) * w1b_ref[2:3, :] - b1_ref[0:1, :]))
        gblk = g_ref[...].reshape(P * q, TW)
        h = jnp.dot(gblk, w1_ref[...], preferred_element_type=jnp.float32)
        h3 = h.reshape(P, q, 33) - ccomb[None, :, :]
        h3 = jnp.maximum(h3, 0.0)
        hmax = jnp.max(h3, axis=0)
        h2 = jnp.dot(h3.reshape(P * q, 33), w2a_ref[...],
                     preferred_element_type=jnp.float32)
        hb = jnp.dot(hmax, w2b_ref[...],
                     preferred_element_type=jnp.float32) + b2_ref[0:1, :]
        h23 = jnp.maximum(h2.reshape(P, q, 128) + hb[None, :, :], 0.0)
        pooled_ref[...] = jnp.max(h23, axis=0)

    @pl.when(pid >= 20)
    def _pad():
        pooled_ref[...] = jnp.zeros_like(pooled_ref)


def _run_pfn(g3, new_yx, nb_r, ny_r, nx_r, w1pad, w1b, b1r, w2a, w2b, b2r):
    q = 1000
    grid = PFN_GRID
    cap19 = lambda i: jnp.minimum(i, 19)
    return pl.pallas_call(
        _pfn_kernel,
        grid=(grid,),
        in_specs=[
            pl.BlockSpec((P, q, TW), lambda i: (0, jnp.minimum(i, 19), 0)),
            pl.BlockSpec((q, 2), lambda i: (jnp.minimum(i, 19), 0)),
            pl.BlockSpec((1, 1, q), lambda i: (jnp.minimum(i, 22), 0, 0)),
            pl.BlockSpec((1, 1, q), lambda i: (jnp.minimum(i, 22), 0, 0)),
            pl.BlockSpec((1, 1, q), lambda i: (jnp.minimum(i, 22), 0, 0)),
            pl.BlockSpec((TW, 33), lambda i: (0, 0)),
            pl.BlockSpec((3, 33), lambda i: (0, 0)),
            pl.BlockSpec((1, 33), lambda i: (0, 0)),
            pl.BlockSpec((33, 128), lambda i: (0, 0)),
            pl.BlockSpec((33, 128), lambda i: (0, 0)),
            pl.BlockSpec((1, 128), lambda i: (0, 0)),
        ],
        out_specs=[
            pl.BlockSpec((q, 128), lambda i: (i, 0)),
            pl.BlockSpec((1, 1, q), lambda i: (i, 0, 0)),
        ],
        out_shape=[
            jax.ShapeDtypeStruct((POOLED_ROWS, 128), jnp.float32),
            jax.ShapeDtypeStruct((PFN_GRID, 1, q), jnp.int32),
        ],
    )(g3, new_yx, nb_r, ny_r, nx_r, w1pad, w1b, b1r, w2a, w2b, b2r)


def _make_densify():
    chunk = 256
    npair = SLAB // (2 * chunk)
    nkc = M // 16
    mesh = plsc.VectorSubcoreMesh(core_axis_name="c", subcore_axis_name="s")

    @functools.partial(
        pl.kernel, mesh=mesh,
        out_type=jax.ShapeDtypeStruct((DENSE_ROWS, 128), jnp.float32),
        scratch_types=[
            pltpu.VMEM((M,), jnp.int32),
            pltpu.VMEM((SLAB,), jnp.int32),
            pltpu.VMEM((chunk, 128), jnp.float32),
            pltpu.VMEM((chunk, 128), jnp.float32),
            pltpu.SemaphoreType.DMA,
            pltpu.SemaphoreType.DMA,
            pltpu.SemaphoreType.DMA,
            pltpu.SemaphoreType.DMA,
        ],
    )
    def k(keys_hbm, pooled_hbm, dense_hbm, keys_v, owner_v, buf0, buf1,
          g0, g1, w0, w1):
        wid = lax.axis_index("s") * 2 + lax.axis_index("c")
        base = wid * SLAB
        lane = lax.iota(jnp.int32, 16)

        pltpu.sync_copy(keys_hbm, keys_v)

        def initb(i, _):
            owner_v[pl.ds(i * 16, 16)] = (
                M + wid * PAD_PER_W + ((lane + i * 16) & (PAD_PER_W - 1)))
            return _

        lax.fori_loop(0, SLAB // 16, initb, 0)

        def scan(c, _):
            kk = keys_v[pl.ds(c * 16, 16)]
            loc = kk - base
            for j in range(16):
                loc_j = loc[j]

                @pl.when((loc_j >= 0) & (loc_j < SLAB))
                def _upd():
                    off = loc_j & ~15
                    cur = owner_v[pl.ds(off, 16)]
                    owner_v[pl.ds(off, 16)] = jnp.where(
                        lane == (loc_j & 15), c * 16 + j, cur)

            return _

        lax.fori_loop(0, nkc, scan, 0)

        def emit(i, _):
            c0 = 2 * i * chunk
            c1 = c0 + chunk
            h0 = pltpu.async_copy(
                pooled_hbm.at[owner_v.at[pl.ds(c0, chunk)]], buf0, g0)
            h1 = pltpu.async_copy(
                pooled_hbm.at[owner_v.at[pl.ds(c1, chunk)]], buf1, g1)
            h0.wait()
            o0 = pltpu.async_copy(
                buf0, dense_hbm.at[pl.ds(base + c0, chunk)], w0)
            h1.wait()
            o1 = pltpu.async_copy(
                buf1, dense_hbm.at[pl.ds(base + c1, chunk)], w1)
            o0.wait()
            o1.wait()
            return _

        lax.fori_loop(0, npair, emit, 0)

    return k


def _tr_kernel(in_ref, out_ref):
    for r in range(8):
        out_ref[0, :, r, :] = in_ref[pl.ds(r * 360, 360), :].T


def _run_transpose(dense):
    rows = 2880
    return pl.pallas_call(
        _tr_kernel,
        grid=(2, 45),
        in_specs=[pl.BlockSpec((rows, 128), lambda b, g: (b * 45 + g, 0))],
        out_specs=pl.BlockSpec((1, 128, 8, 360), lambda b, g: (b, 0, g, 0)),
        out_shape=jax.ShapeDtypeStruct((2, 128, 360, 360), jnp.float32),
    )(dense)


def kernel(voxel_features, sp_zyx, key_indices, new_b, new_yx, bev,
           W1, g1, b1, W2, g2, b2):
    del bev
    w1f = W1 * g1[None, :]
    w2f = W2 * g2[None, :]
    w1pad = jnp.zeros((TW, 33), jnp.float32).at[:67, :].set(w1f)
    w1b = w1f[64:67, :]
    w2a = w2f[:33, :]
    w2b = w2f[33:, :]
    b1r = b1.reshape(1, 33)
    b2r = b2.reshape(1, 128)

    sp_zyx = sp_zyx.astype(jnp.int32)
    key_indices = key_indices.astype(jnp.int32)
    new_b = new_b.astype(jnp.int32)
    new_yx = new_yx.astype(jnp.int32)

    padq = PFN_GRID * 1000 - M
    nb_p = jnp.concatenate([new_b, jnp.full((padq,), 600, jnp.int32)])
    ny_p = jnp.concatenate([new_yx[:, 0], jnp.zeros((padq,), jnp.int32)])
    nx_p = jnp.concatenate([new_yx[:, 1], jnp.zeros((padq,), jnp.int32)])
    nb_r = nb_p.reshape(PFN_GRID, 1, 1000)
    ny_r = ny_p.reshape(PFN_GRID, 1, 1000)
    nx_r = nx_p.reshape(PFN_GRID, 1, 1000)

    table = _build_table(voxel_features, sp_zyx)
    kit = key_indices.T.reshape(M * P)
    g = _make_gather()(table, kit)
    g3 = g.reshape(P, M, TW)
    pooled, keys_r = _run_pfn(g3, new_yx, nb_r, ny_r, nx_r,
                              w1pad, w1b, b1r, w2a, w2b, b2r)
    keys = keys_r[:20].reshape(M)
    dense = _make_densify()(keys, pooled)
    return _run_transpose(dense)


# speedup vs baseline: 1.0366x; 1.0366x over previous
import functools

import jax
import jax.numpy as jnp
from jax import lax
from jax.experimental import pallas as pl
from jax.experimental.pallas import tpu as pltpu
from jax.experimental.pallas import tpu_sc as plsc

N_VOX = 100000
M = 20000
P = 16
TW = 128
NWORK = 32
SLAB = 8192
DENSE_ROWS = NWORK * SLAB
NPIX = 2 * 360 * 360
PAD_PER_W = 64
POOLED_ROWS = M + NWORK * PAD_PER_W
PFN_GRID = 23

VSX = 150.4 / 1440.0
VSZ = 6.0 / 40.0
VNX = 150.4 / 360.0
PMIN = -75.2
PMINZ = -2.0


def _table_kernel(vf_ref, zyx_ref, t_ref):
    vf = vf_ref[...]
    z = (zyx_ref[:, 0:1] % 40).astype(jnp.float32)
    y = zyx_ref[:, 1:2].astype(jnp.float32)
    x = zyx_ref[:, 2:3].astype(jnp.float32)
    cx = (x + 0.5) * VSX + PMIN
    cy = (y + 0.5) * VSX + PMIN
    cz = (z + 0.5) * VSZ + PMINZ
    pad = jnp.zeros((vf.shape[0], TW - 67), jnp.float32)
    t_ref[...] = jnp.concatenate([vf, cx, cy, cz, pad], axis=1)


def _build_table(voxel_features, sp_zyx):
    blk = 4000
    grid = N_VOX // blk
    return pl.pallas_call(
        _table_kernel,
        grid=(grid,),
        in_specs=[
            pl.BlockSpec((blk, 64), lambda i: (i, 0)),
            pl.BlockSpec((blk, 3), lambda i: (i, 0)),
        ],
        out_specs=pl.BlockSpec((blk, TW), lambda i: (i, 0)),
        out_shape=jax.ShapeDtypeStruct((N_VOX, TW), jnp.float32),
    )(voxel_features, sp_zyx)


def _make_gather():
    rows_per_w = (M * P) // NWORK
    chunk = 400
    npair = rows_per_w // (2 * chunk)
    mesh = plsc.VectorSubcoreMesh(core_axis_name="c", subcore_axis_name="s")

    @functools.partial(
        pl.kernel, mesh=mesh,
        out_type=jax.ShapeDtypeStruct((M * P, TW), jnp.float32),
        scratch_types=[
            pltpu.VMEM((rows_per_w,), jnp.int32),
            pltpu.VMEM((chunk, TW), jnp.float32),
            pltpu.VMEM((chunk, TW), jnp.float32),
            pltpu.SemaphoreType.DMA,
            pltpu.SemaphoreType.DMA,
            pltpu.SemaphoreType.DMA,
            pltpu.SemaphoreType.DMA,
        ],
    )
    def k(table_hbm, idx_hbm, out_hbm, idx_v, buf0, buf1, g0, g1, w0, w1):
        wid = lax.axis_index("s") * 2 + lax.axis_index("c")
        base = wid * rows_per_w
        pltpu.sync_copy(idx_hbm.at[pl.ds(base, rows_per_w)], idx_v)

        def body(i, _):
            c0 = base + 2 * i * chunk
            c1 = c0 + chunk
            h0 = pltpu.async_copy(
                table_hbm.at[idx_v.at[pl.ds(2 * i * chunk, chunk)]], buf0, g0)
            h1 = pltpu.async_copy(
                table_hbm.at[idx_v.at[pl.ds((2 * i + 1) * chunk, chunk)]],
                buf1, g1)
            h0.wait()
            o0 = pltpu.async_copy(buf0, out_hbm.at[pl.ds(c0, chunk)], w0)
            h1.wait()
            o1 = pltpu.async_copy(buf1, out_hbm.at[pl.ds(c1, chunk)], w1)
            o0.wait()
            o1.wait()
            return _

        lax.fori_loop(0, npair, body, 0)
        rem = rows_per_w - npair * 2 * chunk
        if rem:
            pltpu.async_copy(
                table_hbm.at[idx_v.at[pl.ds(npair * 2 * chunk, rem)]],
                buf0.at[pl.ds(0, rem)], g0).wait()
            pltpu.sync_copy(buf0.at[pl.ds(0, rem)],
                            out_hbm.at[pl.ds(base + npair * 2 * chunk, rem)])

    return k


def _pfn_kernel(g_ref, yx_ref, nb_ref, ny_ref, nx_ref,
                w1_ref, w1b_ref, b1_ref, w2a_ref, w2b_ref, b2_ref,
                pooled_ref, keys_ref):
    pid = pl.program_id(0)
    q = g_ref.shape[1]

    b_row = nb_ref[0]
    y_row = ny_ref[0]
    x_row = nx_ref[0]
    keys_ref[0] = (b_row * 360 + y_row) * 360 + x_row

    @pl.when(pid < 20)
    def _compute():
        qx = (yx_ref[:, 1:2].astype(jnp.float32) + 0.5) * VNX + PMIN
        qy = (yx_ref[:, 0:1].astype(jnp.float32) + 0.5) * VNX + PMIN
        ccomb = (qx * w1b_ref[0:1, :] + qy * w1b_ref[1:2, :]
                 + ((0.5 * 6.0 + PMINZ) * w1b_ref[2:3, :] - b1_ref[0:1, :]))
        gblk = g_ref[...].reshape(P * q, TW)
        h = jnp.dot(gblk, w1_ref[...], preferred_element_type=jnp.float32)
        h3 = h.reshape(P, q, 33) - ccomb[None, :, :]
        h3 = jnp.maximum(h3, 0.0)
        hmax = jnp.max(h3, axis=0)
        h2 = jnp.dot(h3.reshape(P * q, 33), w2a_ref[...],
                     preferred_element_type=jnp.float32)
        hb = jnp.dot(hmax, w2b_ref[...],
                     preferred_element_type=jnp.float32) + b2_ref[0:1, :]
        h23 = jnp.maximum(h2.reshape(P, q, 128) + hb[None, :, :], 0.0)
        pooled_ref[...] = jnp.max(h23, axis=0)

    @pl.when(pid >= 20)
    def _pad():
        pooled_ref[...] = jnp.zeros_like(pooled_ref)


def _run_pfn(g3, new_yx, nb_r, ny_r, nx_r, w1pad, w1b, b1r, w2a, w2b, b2r):
    q = 1000
    grid = PFN_GRID
    cap19 = lambda i: jnp.minimum(i, 19)
    return pl.pallas_call(
        _pfn_kernel,
        grid=(grid,),
        in_specs=[
            pl.BlockSpec((P, q, TW), lambda i: (0, jnp.minimum(i, 19), 0)),
            pl.BlockSpec((q, 2), lambda i: (jnp.minimum(i, 19), 0)),
            pl.BlockSpec((1, 1, q), lambda i: (jnp.minimum(i, 22), 0, 0)),
            pl.BlockSpec((1, 1, q), lambda i: (jnp.minimum(i, 22), 0, 0)),
            pl.BlockSpec((1, 1, q), lambda i: (jnp.minimum(i, 22), 0, 0)),
            pl.BlockSpec((TW, 33), lambda i: (0, 0)),
            pl.BlockSpec((3, 33), lambda i: (0, 0)),
            pl.BlockSpec((1, 33), lambda i: (0, 0)),
            pl.BlockSpec((33, 128), lambda i: (0, 0)),
            pl.BlockSpec((33, 128), lambda i: (0, 0)),
            pl.BlockSpec((1, 128), lambda i: (0, 0)),
        ],
        out_specs=[
            pl.BlockSpec((q, 128), lambda i: (i, 0)),
            pl.BlockSpec((1, 1, q), lambda i: (i, 0, 0)),
        ],
        out_shape=[
            jax.ShapeDtypeStruct((POOLED_ROWS, 128), jnp.float32),
            jax.ShapeDtypeStruct((PFN_GRID, 1, q), jnp.int32),
        ],
    )(g3, new_yx, nb_r, ny_r, nx_r, w1pad, w1b, b1r, w2a, w2b, b2r)


def _make_densify():
    chunk = 256
    npair = SLAB // (2 * chunk)
    nkc = M // 16
    mesh = plsc.VectorSubcoreMesh(core_axis_name="c", subcore_axis_name="s")

    @functools.partial(
        pl.kernel, mesh=mesh,
        out_type=jax.ShapeDtypeStruct((DENSE_ROWS, 128), jnp.float32),
        scratch_types=[
            pltpu.VMEM((M,), jnp.int32),
            pltpu.VMEM((SLAB,), jnp.int32),
            pltpu.VMEM((chunk, 128), jnp.float32),
            pltpu.VMEM((chunk, 128), jnp.float32),
            pltpu.SemaphoreType.DMA,
            pltpu.SemaphoreType.DMA,
            pltpu.SemaphoreType.DMA,
            pltpu.SemaphoreType.DMA,
        ],
    )
    def k(keys_hbm, pooled_hbm, dense_hbm, keys_v, owner_v, buf0, buf1,
          g0, g1, w0, w1):
        wid = lax.axis_index("s") * 2 + lax.axis_index("c")
        base = wid * SLAB
        lane = lax.iota(jnp.int32, 16)

        pltpu.sync_copy(keys_hbm, keys_v)

        def initb(i, _):
            owner_v[pl.ds(i * 16, 16)] = (
                M + wid * PAD_PER_W + ((lane + i * 16) & (PAD_PER_W - 1)))
            return _

        lax.fori_loop(0, SLAB // 16, initb, 0)

        def scan(c, _):
            kk = keys_v[pl.ds(c * 16, 16)]
            loc = kk - base
            for j in range(16):
                loc_j = loc[j]

                @pl.when((loc_j >= 0) & (loc_j < SLAB))
                def _upd():
                    off = loc_j & ~15
                    cur = owner_v[pl.ds(off, 16)]
                    owner_v[pl.ds(off, 16)] = jnp.where(
                        lane == (loc_j & 15), c * 16 + j, cur)

            return _

        lax.fori_loop(0, nkc, scan, 0)

        def emit(i, _):
            c0 = 2 * i * chunk
            c1 = c0 + chunk
            h0 = pltpu.async_copy(
                pooled_hbm.at[owner_v.at[pl.ds(c0, chunk)]], buf0, g0)
            h1 = pltpu.async_copy(
                pooled_hbm.at[owner_v.at[pl.ds(c1, chunk)]], buf1, g1)
            h0.wait()
            o0 = pltpu.async_copy(
                buf0, dense_hbm.at[pl.ds(base + c0, chunk)], w0)
            h1.wait()
            o1 = pltpu.async_copy(
                buf1, dense_hbm.at[pl.ds(base + c1, chunk)], w1)
            o0.wait()
            o1.wait()
            return _

        lax.fori_loop(0, npair, emit, 0)

    return k


def _tr_kernel(in_ref, out_ref):
    for r in range(8):
        out_ref[0, :, r, :] = in_ref[pl.ds(r * 360, 360), :].T


def _run_transpose(dense):
    rows = 2880
    return pl.pallas_call(
        _tr_kernel,
        grid=(2, 45),
        in_specs=[pl.BlockSpec((rows, 128), lambda b, g: (b * 45 + g, 0))],
        out_specs=pl.BlockSpec((1, 128, 8, 360), lambda b, g: (b, 0, g, 0)),
        out_shape=jax.ShapeDtypeStruct((2, 128, 360, 360), jnp.float32),
    )(dense)


def kernel(voxel_features, sp_zyx, key_indices, new_b, new_yx, bev,
           W1, g1, b1, W2, g2, b2):
    del bev
    w1f = W1 * g1[None, :]
    w2f = W2 * g2[None, :]
    w1pad = jnp.zeros((TW, 33), jnp.float32).at[:67, :].set(w1f)
    w1b = w1f[64:67, :]
    w2a = w2f[:33, :]
    w2b = w2f[33:, :]
    b1r = b1.reshape(1, 33)
    b2r = b2.reshape(1, 128)

    sp_zyx = sp_zyx.astype(jnp.int32)
    key_indices = key_indices.astype(jnp.int32)
    new_b = new_b.astype(jnp.int32)
    new_yx = new_yx.astype(jnp.int32)

    padq = PFN_GRID * 1000 - M
    nb_p = jnp.concatenate([new_b, jnp.full((padq,), 600, jnp.int32)])
    ny_p = jnp.concatenate([new_yx[:, 0], jnp.zeros((padq,), jnp.int32)])
    nx_p = jnp.concatenate([new_yx[:, 1], jnp.zeros((padq,), jnp.int32)])
    nb_r = nb_p.reshape(PFN_GRID, 1, 1000)
    ny_r = ny_p.reshape(PFN_GRID, 1, 1000)
    nx_r = nx_p.reshape(PFN_GRID, 1, 1000)

    table = _build_table(voxel_features, sp_zyx)
    kit = key_indices.T.reshape(M * P)
    g = _make_gather()(table, kit)
    g3 = g.reshape(P, M, TW)
    pooled, keys_r = _run_pfn(g3, new_yx, nb_r, ny_r, nx_r,
                              w1pad, w1b, b1r, w2a, w2b, b2r)
    keys = keys_r[:20].reshape(M)
    dense = _make_densify()(keys, pooled)
    return _run_transpose(dense)
